# baseline (speedup 1.0000x reference)
import functools

import jax
import jax.numpy as jnp
from jax import lax
from jax.experimental import pallas as pl
from jax.experimental.pallas import tpu as pltpu

BR = 512


def _tobf16_body(A_ref, Ab_ref):
    Ab_ref[...] = A_ref[...].astype(jnp.bfloat16)


def _tobf16(A):
    N = A.shape[0]
    return pl.pallas_call(
        _tobf16_body,
        grid=(N // BR,),
        in_specs=[pl.BlockSpec((BR, N), lambda i: (i, 0))],
        out_specs=pl.BlockSpec((BR, N), lambda i: (i, 0)),
        out_shape=jax.ShapeDtypeStruct((N, N), jnp.bfloat16),
    )(A)


def _gat_body(Z_ref, A_ref, W_ref, as_ref, ar_ref, out_ref, C_ref,
              H_ref, Hb_ref, f1_ref, f2_ref, Eb_ref):
    i = pl.program_id(0)

    @pl.when(i == 0)
    def _prologue():
        H = jnp.dot(Z_ref[...], W_ref[...], preferred_element_type=jnp.float32)
        H_ref[...] = H
        Hb_ref[...] = H.astype(jnp.bfloat16)
        f1_ref[...] = 0.5 * lax.dot_general(
            H, as_ref[...], (((1,), (1,)), ((), ())),
            preferred_element_type=jnp.float32)
        f2_ref[...] = 0.5 * lax.dot_general(
            ar_ref[...], H, (((1,), (1,)), ((), ())),
            preferred_element_type=jnp.float32)

    f1b = f1_ref[pl.ds(i * BR, BR), :]
    e = 0.5 * jnp.tanh(f1b + f2_ref[...]) + 0.5
    E = A_ref[...] * e
    Eb_ref[...] = E.astype(jnp.bfloat16)
    r = 1.0 / (jnp.sum(E, axis=1, keepdims=True) + 1e-8)
    Eb = Eb_ref[...]
    C_ref[...] = Eb.astype(jnp.float32) * r
    out_ref[...] = jnp.dot(Eb, Hb_ref[...],
                           preferred_element_type=jnp.float32) * r


def _gat(Z, A, W, a_s, a_r):
    N = A.shape[0]
    din, dout = W.shape
    out, C = pl.pallas_call(
        _gat_body,
        grid=(N // BR,),
        in_specs=[
            pl.BlockSpec((N, din), lambda i: (0, 0)),
            pl.BlockSpec((BR, N), lambda i: (i, 0)),
            pl.BlockSpec((din, dout), lambda i: (0, 0)),
            pl.BlockSpec((1, dout), lambda i: (0, 0)),
            pl.BlockSpec((1, dout), lambda i: (0, 0)),
        ],
        out_specs=(
            pl.BlockSpec((BR, dout), lambda i: (i, 0)),
            pl.BlockSpec((BR, N), lambda i: (i, 0)),
        ),
        out_shape=(
            jax.ShapeDtypeStruct((N, dout), jnp.float32),
            jax.ShapeDtypeStruct((N, N), jnp.float32),
        ),
        scratch_shapes=[
            pltpu.VMEM((N, dout), jnp.float32),
            pltpu.VMEM((N, dout), jnp.bfloat16),
            pltpu.VMEM((N, 1), jnp.float32),
            pltpu.VMEM((1, N), jnp.float32),
            pltpu.VMEM((BR, N), jnp.bfloat16),
        ],
    )(Z, A, W, a_s.reshape(1, -1), a_r.reshape(1, -1))
    return out, C


def _gat_an_body(Z_ref, A_ref, W_ref, as_ref, ar_ref, P_ref,
                 out_ref, C_ref, An_ref, H_ref, Hb_ref, f1_ref, f2_ref,
                 Eb_ref):
    i = pl.program_id(0)

    @pl.when(i == 0)
    def _prologue():
        H = jnp.dot(Z_ref[...], W_ref[...], preferred_element_type=jnp.float32)
        H_ref[...] = H
        Hb_ref[...] = H.astype(jnp.bfloat16)
        f1_ref[...] = 0.5 * lax.dot_general(
            H, as_ref[...], (((1,), (1,)), ((), ())),
            preferred_element_type=jnp.float32)
        f2_ref[...] = 0.5 * lax.dot_general(
            ar_ref[...], H, (((1,), (1,)), ((), ())),
            preferred_element_type=jnp.float32)

    A_blk = A_ref[...]
    f1b = f1_ref[pl.ds(i * BR, BR), :]
    e = 0.5 * jnp.tanh(f1b + f2_ref[...]) + 0.5
    E = A_blk * e
    Eb_ref[...] = E.astype(jnp.bfloat16)
    r = 1.0 / (jnp.sum(E, axis=1, keepdims=True) + 1e-8)
    Eb = Eb_ref[...]
    C_ref[...] = Eb.astype(jnp.float32) * r
    out_ref[...] = jnp.dot(Eb, Hb_ref[...],
                           preferred_element_type=jnp.float32) * r

    AP = jnp.dot(A_blk, P_ref[...].astype(jnp.bfloat16),
                 preferred_element_type=jnp.float32)
    Pb = P_ref[pl.ds(i * BR, BR), :]
    contrib = lax.dot_general(Pb, AP, (((0,), (0,)), ((), ())),
                              preferred_element_type=jnp.float32)

    @pl.when(i == 0)
    def _init():
        An_ref[...] = contrib

    @pl.when(i > 0)
    def _acc():
        An_ref[...] += contrib


def _gat_with_an(Z, A, W, a_s, a_r, P):
    N = A.shape[0]
    din, dout = W.shape
    c = P.shape[1]
    out, C, An = pl.pallas_call(
        _gat_an_body,
        grid=(N // BR,),
        in_specs=[
            pl.BlockSpec((N, din), lambda i: (0, 0)),
            pl.BlockSpec((BR, N), lambda i: (i, 0)),
            pl.BlockSpec((din, dout), lambda i: (0, 0)),
            pl.BlockSpec((1, dout), lambda i: (0, 0)),
            pl.BlockSpec((1, dout), lambda i: (0, 0)),
            pl.BlockSpec((N, c), lambda i: (0, 0)),
        ],
        out_specs=(
            pl.BlockSpec((BR, dout), lambda i: (i, 0)),
            pl.BlockSpec((BR, N), lambda i: (i, 0)),
            pl.BlockSpec((c, c), lambda i: (0, 0)),
        ),
        out_shape=(
            jax.ShapeDtypeStruct((N, dout), jnp.float32),
            jax.ShapeDtypeStruct((N, N), jnp.float32),
            jax.ShapeDtypeStruct((c, c), jnp.float32),
        ),
        scratch_shapes=[
            pltpu.VMEM((N, dout), jnp.float32),
            pltpu.VMEM((N, dout), jnp.bfloat16),
            pltpu.VMEM((N, 1), jnp.float32),
            pltpu.VMEM((1, N), jnp.float32),
            pltpu.VMEM((BR, N), jnp.bfloat16),
        ],
    )(Z, A, W, a_s.reshape(1, -1), a_r.reshape(1, -1), P)
    return out, C, An


BRM = 128
BRL = 256


def _gat_l1_body(Z_ref, A_ref, W_ref, as_ref, ar_ref, out_ref, C_ref, Ab_ref,
                 H_ref, Hb_ref, f1_ref, f2_ref, Eb_ref):
    i = pl.program_id(0)

    @pl.when(i == 0)
    def _prologue():
        H = jnp.dot(Z_ref[...], W_ref[...], preferred_element_type=jnp.float32)
        H_ref[...] = H
        Hb_ref[...] = H.astype(jnp.bfloat16)
        f1_ref[...] = 0.5 * lax.dot_general(
            H, as_ref[...], (((1,), (1,)), ((), ())),
            preferred_element_type=jnp.float32)
        f2_ref[...] = 0.5 * lax.dot_general(
            ar_ref[...], H, (((1,), (1,)), ((), ())),
            preferred_element_type=jnp.float32)

    A_blk = A_ref[...]
    Ab_ref[...] = A_blk.astype(jnp.bfloat16)
    f1b = f1_ref[pl.ds(i * BRL, BRL), :]
    e = 0.5 * jnp.tanh(f1b + f2_ref[...]) + 0.5
    E = A_blk * e
    Eb_ref[...] = E.astype(jnp.bfloat16)
    r = 1.0 / (jnp.sum(E, axis=1, keepdims=True) + 1e-8)
    Eb = Eb_ref[...]
    C_ref[...] = Eb.astype(jnp.float32) * r
    out_ref[...] = jnp.dot(Eb, Hb_ref[...],
                           preferred_element_type=jnp.float32) * r


def _gat_l1(Z, A, W, a_s, a_r):
    N = A.shape[0]
    din, dout = W.shape
    out, C, Ab = pl.pallas_call(
        _gat_l1_body,
        grid=(N // BRL,),
        in_specs=[
            pl.BlockSpec((N, din), lambda i: (0, 0)),
            pl.BlockSpec((BRL, N), lambda i: (i, 0)),
            pl.BlockSpec((din, dout), lambda i: (0, 0)),
            pl.BlockSpec((1, dout), lambda i: (0, 0)),
            pl.BlockSpec((1, dout), lambda i: (0, 0)),
        ],
        out_specs=(
            pl.BlockSpec((BRL, dout), lambda i: (i, 0)),
            pl.BlockSpec((BRL, N), lambda i: (i, 0)),
            pl.BlockSpec((BRL, N), lambda i: (i, 0)),
        ),
        out_shape=(
            jax.ShapeDtypeStruct((N, dout), jnp.float32),
            jax.ShapeDtypeStruct((N, N), jnp.float32),
            jax.ShapeDtypeStruct((N, N), jnp.bfloat16),
        ),
        scratch_shapes=[
            pltpu.VMEM((N, dout), jnp.float32),
            pltpu.VMEM((N, dout), jnp.bfloat16),
            pltpu.VMEM((N, 1), jnp.float32),
            pltpu.VMEM((1, N), jnp.float32),
            pltpu.VMEM((BRL, N), jnp.bfloat16),
        ],
    )(Z, A, W, a_s.reshape(1, -1), a_r.reshape(1, -1))
    return out, C, Ab


def _mega_enc_body(Z1_ref, Ab_ref, W1_ref, s1_ref, r1_ref,
                   W2_ref, s2_ref, r2_ref,
                   C2_ref, C3_ref, Ze_ref,
                   Hb_ref, f1_ref, f2_ref, Xa_ref, Eb_ref):
    p = pl.program_id(0)
    i = pl.program_id(1)

    @pl.when(jnp.logical_and(p == 0, i == 0))
    def _pro0():
        H = jnp.dot(Z1_ref[...], W1_ref[...].astype(jnp.bfloat16),
                    preferred_element_type=jnp.float32)
        Hb_ref[...] = H.astype(jnp.bfloat16)
        f1_ref[...] = 0.5 * lax.dot_general(
            H, s1_ref[...], (((1,), (1,)), ((), ())),
            preferred_element_type=jnp.float32)
        f2_ref[...] = 0.5 * lax.dot_general(
            r1_ref[...], H, (((1,), (1,)), ((), ())),
            preferred_element_type=jnp.float32)

    @pl.when(jnp.logical_and(p == 1, i == 0))
    def _pro1():
        H = jnp.dot(Xa_ref[...], W2_ref[...].astype(jnp.bfloat16),
                    preferred_element_type=jnp.float32)
        Hb_ref[...] = H.astype(jnp.bfloat16)
        f1_ref[...] = 0.5 * lax.dot_general(
            H, s2_ref[...], (((1,), (1,)), ((), ())),
            preferred_element_type=jnp.float32)
        f2_ref[...] = 0.5 * lax.dot_general(
            r2_ref[...], H, (((1,), (1,)), ((), ())),
            preferred_element_type=jnp.float32)

    Ablk = Ab_ref[pl.ds(i * BRM, BRM), :]
    f1b = f1_ref[pl.ds(i * BRM, BRM), :]
    e = 0.5 * jnp.tanh(f1b + f2_ref[...]) + 0.5
    E = Ablk * e
    Eb_ref[...] = E.astype(jnp.bfloat16)
    r = 1.0 / (jnp.sum(E, axis=1, keepdims=True) + 1e-8)
    Eb = Eb_ref[...]
    Cblk = Eb.astype(jnp.float32) * r
    out = jnp.dot(Eb, Hb_ref[...], preferred_element_type=jnp.float32) * r

    @pl.when(p == 0)
    def _st0():
        C2_ref[...] = Cblk
        Xa_ref[pl.ds(i * BRM, BRM), :] = out.astype(jnp.bfloat16)

    @pl.when(p == 1)
    def _st1():
        C3_ref[...] = Cblk
        Ze_ref[...] = out[:, :64]


def _mega_enc(Z1b, Ab, W1p, s1p, r1p, W2p, s2p, r2p):
    N = Ab.shape[0]
    NS = N // BRM

    def _clamp(k):
        return lambda p, i: (jnp.clip((p - k) * NS + i, 0, NS - 1), 0)

    C2, C3, Ze = pl.pallas_call(
        _mega_enc_body,
        grid=(2, NS),
        in_specs=[
            pl.BlockSpec((N, 256), lambda p, i: (0, 0)),
            pl.BlockSpec((N, N), lambda p, i: (0, 0)),
            pl.BlockSpec((256, 256), lambda p, i: (0, 0)),
            pl.BlockSpec((1, 256), lambda p, i: (0, 0)),
            pl.BlockSpec((1, 256), lambda p, i: (0, 0)),
            pl.BlockSpec((256, 256), lambda p, i: (0, 0)),
            pl.BlockSpec((1, 256), lambda p, i: (0, 0)),
            pl.BlockSpec((1, 256), lambda p, i: (0, 0)),
        ],
        out_specs=(
            pl.BlockSpec((BRM, N), _clamp(0)),
            pl.BlockSpec((BRM, N), _clamp(1)),
            pl.BlockSpec((BRM, 64), _clamp(1)),
        ),
        out_shape=(
            jax.ShapeDtypeStruct((N, N), jnp.float32),
            jax.ShapeDtypeStruct((N, N), jnp.float32),
            jax.ShapeDtypeStruct((N, 64), jnp.float32),
        ),
        scratch_shapes=[
            pltpu.VMEM((N, 256), jnp.bfloat16),
            pltpu.VMEM((N, 1), jnp.float32),
            pltpu.VMEM((1, N), jnp.float32),
            pltpu.VMEM((N, 256), jnp.bfloat16),
            pltpu.VMEM((BRM, N), jnp.bfloat16),
        ],
    )(Z1b, Ab, W1p, s1p, r1p, W2p, s2p, r2p)
    return C2, C3, Ze


def _mega_dec_body(Ze_ref, Ab_ref, W0_ref, s0_ref, r0_ref,
                   W1_ref, s1_ref, r1_ref, W2_ref, s2_ref, r2_ref, Pb_ref,
                   C4_ref, C5_ref, C6_ref, Xh_ref, An_ref,
                   Hb_ref, f1_ref, f2_ref, Xa_ref, Xb_ref, Eb_ref):
    p = pl.program_id(0)
    i = pl.program_id(1)

    @pl.when(jnp.logical_and(p == 0, i == 0))
    def _pro0():
        H = jnp.dot(Ze_ref[...], W0_ref[...].astype(jnp.bfloat16),
                    preferred_element_type=jnp.float32)
        Hb_ref[...] = H.astype(jnp.bfloat16)
        f1_ref[...] = 0.5 * lax.dot_general(
            H, s0_ref[...], (((1,), (1,)), ((), ())),
            preferred_element_type=jnp.float32)
        f2_ref[...] = 0.5 * lax.dot_general(
            r0_ref[...], H, (((1,), (1,)), ((), ())),
            preferred_element_type=jnp.float32)

    @pl.when(jnp.logical_and(p == 1, i == 0))
    def _pro1():
        H = jnp.dot(Xa_ref[...], W1_ref[...].astype(jnp.bfloat16),
                    preferred_element_type=jnp.float32)
        Hb_ref[...] = H.astype(jnp.bfloat16)
        f1_ref[...] = 0.5 * lax.dot_general(
            H, s1_ref[...], (((1,), (1,)), ((), ())),
            preferred_element_type=jnp.float32)
        f2_ref[...] = 0.5 * lax.dot_general(
            r1_ref[...], H, (((1,), (1,)), ((), ())),
            preferred_element_type=jnp.float32)

    @pl.when(jnp.logical_and(p == 2, i == 0))
    def _pro2():
        H = jnp.dot(Xb_ref[...], W2_ref[...].astype(jnp.bfloat16),
                    preferred_element_type=jnp.float32)
        Hb_ref[...] = H.astype(jnp.bfloat16)
        f1_ref[...] = 0.5 * lax.dot_general(
            H, s2_ref[...], (((1,), (1,)), ((), ())),
            preferred_element_type=jnp.float32)
        f2_ref[...] = 0.5 * lax.dot_general(
            r2_ref[...], H, (((1,), (1,)), ((), ())),
            preferred_element_type=jnp.float32)

    Ablk = Ab_ref[pl.ds(i * BRM, BRM), :]
    f1b = f1_ref[pl.ds(i * BRM, BRM), :]
    e = 0.5 * jnp.tanh(f1b + f2_ref[...]) + 0.5
    E = Ablk * e
    Eb_ref[...] = E.astype(jnp.bfloat16)
    r = 1.0 / (jnp.sum(E, axis=1, keepdims=True) + 1e-8)
    Eb = Eb_ref[...]
    Cblk = Eb.astype(jnp.float32) * r
    out = jnp.dot(Eb, Hb_ref[...], preferred_element_type=jnp.float32) * r

    @pl.when(p == 0)
    def _st0():
        C4_ref[...] = Cblk
        Xa_ref[pl.ds(i * BRM, BRM), :] = out[:, :128].astype(jnp.bfloat16)
        AP = jnp.dot(Ablk, Pb_ref[...], preferred_element_type=jnp.float32)
        Pf = Pb_ref[pl.ds(i * BRM, BRM), :].astype(jnp.float32)
        contrib = lax.dot_general(Pf, AP, (((0,), (0,)), ((), ())),
                                  preferred_element_type=jnp.float32)

        @pl.when(i == 0)
        def _init():
            An_ref[...] = contrib

        @pl.when(i > 0)
        def _acc():
            An_ref[...] += contrib

    @pl.when(p == 1)
    def _st1():
        C5_ref[...] = Cblk
        Xb_ref[pl.ds(i * BRM, BRM), :] = out.astype(jnp.bfloat16)

    @pl.when(p == 2)
    def _st2():
        C6_ref[...] = Cblk
        Xh_ref[...] = out


def _mega_dec(Ze, Ab, W0p, s0p, r0p, W1p, s1p, r1p, W2p, s2p, r2p, Pb):
    N = Ab.shape[0]
    NS = N // BRM

    def _clamp(k):
        return lambda p, i: (jnp.clip((p - k) * NS + i, 0, NS - 1), 0)

    C4, C5, C6, Xh, An64 = pl.pallas_call(
        _mega_dec_body,
        grid=(3, NS),
        in_specs=[
            pl.BlockSpec((N, 64), lambda p, i: (0, 0)),
            pl.BlockSpec((N, N), lambda p, i: (0, 0)),
            pl.BlockSpec((64, 256), lambda p, i: (0, 0)),
            pl.BlockSpec((1, 256), lambda p, i: (0, 0)),
            pl.BlockSpec((1, 256), lambda p, i: (0, 0)),
            pl.BlockSpec((128, 256), lambda p, i: (0, 0)),
            pl.BlockSpec((1, 256), lambda p, i: (0, 0)),
            pl.BlockSpec((1, 256), lambda p, i: (0, 0)),
            pl.BlockSpec((256, 256), lambda p, i: (0, 0)),
            pl.BlockSpec((1, 256), lambda p, i: (0, 0)),
            pl.BlockSpec((1, 256), lambda p, i: (0, 0)),
            pl.BlockSpec((N, 64), lambda p, i: (0, 0)),
        ],
        out_specs=(
            pl.BlockSpec((BRM, N), _clamp(0)),
            pl.BlockSpec((BRM, N), _clamp(1)),
            pl.BlockSpec((BRM, N), _clamp(2)),
            pl.BlockSpec((BRM, 256), _clamp(2)),
            pl.BlockSpec((64, 64), lambda p, i: (0, 0)),
        ),
        out_shape=(
            jax.ShapeDtypeStruct((N, N), jnp.float32),
            jax.ShapeDtypeStruct((N, N), jnp.float32),
            jax.ShapeDtypeStruct((N, N), jnp.float32),
            jax.ShapeDtypeStruct((N, 256), jnp.float32),
            jax.ShapeDtypeStruct((64, 64), jnp.float32),
        ),
        scratch_shapes=[
            pltpu.VMEM((N, 256), jnp.bfloat16),
            pltpu.VMEM((N, 1), jnp.float32),
            pltpu.VMEM((1, N), jnp.float32),
            pltpu.VMEM((N, 128), jnp.bfloat16),
            pltpu.VMEM((N, 256), jnp.bfloat16),
            pltpu.VMEM((BRM, N), jnp.bfloat16),
        ],
    )(Ze, Ab, W0p, s0p, r0p, W1p, s1p, r1p, W2p, s2p, r2p, Pb)
    return C4, C5, C6, Xh, An64


def _ahat_body(Z_ref, g_ref, b_ref, out_ref):
    i = pl.program_id(0)
    Zb = Z_ref[pl.ds(i * BR, BR), :]
    G = lax.dot_general(Zb, Z_ref[...], (((1,), (1,)), ((), ())),
                        preferred_element_type=jnp.float32)
    mu = jnp.mean(G, axis=1, keepdims=True)
    d = G - mu
    var = jnp.mean(d * d, axis=1, keepdims=True)
    y = d * lax.rsqrt(var + 1e-5) * g_ref[...] + b_ref[...]
    out_ref[...] = 0.5 * jnp.tanh(0.5 * y) + 0.5


def _ahat(Z, g, b):
    N = Z.shape[0]
    h = Z.shape[1]
    return pl.pallas_call(
        _ahat_body,
        grid=(N // BR,),
        in_specs=[
            pl.BlockSpec((N, h), lambda i: (0, 0)),
            pl.BlockSpec((1, N), lambda i: (0, 0)),
            pl.BlockSpec((1, N), lambda i: (0, 0)),
        ],
        out_specs=pl.BlockSpec((BR, N), lambda i: (i, 0)),
        out_shape=jax.ShapeDtypeStruct((N, N), jnp.float32),
    )(Z, g.reshape(1, -1), b.reshape(1, -1))


def _padw(W, din, dout):
    return jnp.pad(W, ((0, din - W.shape[0]), (0, dout - W.shape[1])))


def _padv(v, d):
    return jnp.pad(v, (0, d - v.shape[0])).reshape(1, -1)


def kernel(X, A, params):
    Z1, C1, Ab = _gat_l1(X, A, params['We0'], params['ase0'], params['are0'])

    C2, C3, Z = _mega_enc(
        Z1.astype(jnp.bfloat16), Ab,
        _padw(params['We1'], 256, 256), _padv(params['ase1'], 256),
        _padv(params['are1'], 256),
        _padw(params['We2'], 256, 256), _padv(params['ase2'], 256),
        _padv(params['are2'], 256))
    enc_attn = [C1, C2, C3]

    A_hat = _ahat(Z, params['g_ln'], params['b_ln'])

    P0 = jax.nn.softmax(Z @ params['Wc0'] + params['bc0'], axis=1)
    S0 = jnp.argmax(P0, axis=1)
    Pb = jnp.pad(P0, ((0, 0), (0, 4))).astype(jnp.bfloat16)

    C4, C5, C6, X_hat, An64 = _mega_dec(
        Z.astype(jnp.bfloat16), Ab,
        _padw(params['Wd0'], 64, 256), _padv(params['asd0'], 256),
        _padv(params['ard0'], 256),
        params['Wd1'], _padv(params['asd1'], 256),
        _padv(params['ard1'], 256),
        _padw(params['Wd2'], 256, 256), _padv(params['asd2'], 256),
        _padv(params['ard2'], 256), Pb)
    dec_attn = [C4, C5, C6]
    An1 = An64[:60, :60]

    Xn1 = P0.T @ Z

    P1 = jax.nn.softmax(Xn1 @ params['Wc1'] + params['bc1'], axis=1)
    S1 = jnp.argmax(P1, axis=1)
    Xn2 = P1.T @ Xn1
    An2 = P1.T @ An1 @ P1

    X_all_final = [Z, Xn1, Xn2]
    A_all_final = [A, An1, An2]
    P_all = [P0, P1]
    S_all = [S0, S1]
    return (X_hat, A_hat, X_all_final, A_all_final, P_all, S_all,
            [enc_attn, dec_attn])


# speedup vs baseline: 1.1777x; 1.1777x over previous
import functools

import jax
import jax.numpy as jnp
from jax import lax
from jax.experimental import pallas as pl
from jax.experimental.pallas import tpu as pltpu

BR = 512


def _tobf16_body(A_ref, Ab_ref):
    Ab_ref[...] = A_ref[...].astype(jnp.bfloat16)


def _tobf16(A):
    N = A.shape[0]
    return pl.pallas_call(
        _tobf16_body,
        grid=(N // BR,),
        in_specs=[pl.BlockSpec((BR, N), lambda i: (i, 0))],
        out_specs=pl.BlockSpec((BR, N), lambda i: (i, 0)),
        out_shape=jax.ShapeDtypeStruct((N, N), jnp.bfloat16),
    )(A)


def _gat_body(Z_ref, A_ref, W_ref, as_ref, ar_ref, out_ref, C_ref,
              H_ref, Hb_ref, f1_ref, f2_ref, Eb_ref):
    i = pl.program_id(0)

    @pl.when(i == 0)
    def _prologue():
        H = jnp.dot(Z_ref[...], W_ref[...], preferred_element_type=jnp.float32)
        H_ref[...] = H
        Hb_ref[...] = H.astype(jnp.bfloat16)
        f1_ref[...] = 0.5 * lax.dot_general(
            H, as_ref[...], (((1,), (1,)), ((), ())),
            preferred_element_type=jnp.float32)
        f2_ref[...] = 0.5 * lax.dot_general(
            ar_ref[...], H, (((1,), (1,)), ((), ())),
            preferred_element_type=jnp.float32)

    f1b = f1_ref[pl.ds(i * BR, BR), :]
    e = 0.5 * jnp.tanh(f1b + f2_ref[...]) + 0.5
    Eb_ref[...] = (A_ref[...] * e).astype(jnp.bfloat16)
    Eb = Eb_ref[...]
    r = 1.0 / (jnp.sum(Eb.astype(jnp.float32), axis=1, keepdims=True) + 1e-8)
    C_ref[...] = Eb.astype(jnp.float32) * r
    out_ref[...] = jnp.dot(Eb, Hb_ref[...],
                           preferred_element_type=jnp.float32) * r


def _gat(Z, A, W, a_s, a_r):
    N = A.shape[0]
    din, dout = W.shape
    out, C = pl.pallas_call(
        _gat_body,
        grid=(N // BR,),
        in_specs=[
            pl.BlockSpec((N, din), lambda i: (0, 0)),
            pl.BlockSpec((BR, N), lambda i: (i, 0)),
            pl.BlockSpec((din, dout), lambda i: (0, 0)),
            pl.BlockSpec((1, dout), lambda i: (0, 0)),
            pl.BlockSpec((1, dout), lambda i: (0, 0)),
        ],
        out_specs=(
            pl.BlockSpec((BR, dout), lambda i: (i, 0)),
            pl.BlockSpec((BR, N), lambda i: (i, 0)),
        ),
        out_shape=(
            jax.ShapeDtypeStruct((N, dout), jnp.float32),
            jax.ShapeDtypeStruct((N, N), jnp.float32),
        ),
        scratch_shapes=[
            pltpu.VMEM((N, dout), jnp.float32),
            pltpu.VMEM((N, dout), jnp.bfloat16),
            pltpu.VMEM((N, 1), jnp.float32),
            pltpu.VMEM((1, N), jnp.float32),
            pltpu.VMEM((BR, N), jnp.bfloat16),
        ],
    )(Z, A, W, a_s.reshape(1, -1), a_r.reshape(1, -1))
    return out, C


def _gat_an_body(Z_ref, A_ref, W_ref, as_ref, ar_ref, P_ref,
                 out_ref, C_ref, An_ref, H_ref, Hb_ref, f1_ref, f2_ref,
                 Eb_ref):
    i = pl.program_id(0)

    @pl.when(i == 0)
    def _prologue():
        H = jnp.dot(Z_ref[...], W_ref[...], preferred_element_type=jnp.float32)
        H_ref[...] = H
        Hb_ref[...] = H.astype(jnp.bfloat16)
        f1_ref[...] = 0.5 * lax.dot_general(
            H, as_ref[...], (((1,), (1,)), ((), ())),
            preferred_element_type=jnp.float32)
        f2_ref[...] = 0.5 * lax.dot_general(
            ar_ref[...], H, (((1,), (1,)), ((), ())),
            preferred_element_type=jnp.float32)

    f1b = f1_ref[pl.ds(i * BR, BR), :]
    e = 0.5 * jnp.tanh(f1b + f2_ref[...]) + 0.5
    Eb_ref[...] = (A_ref[...] * e).astype(jnp.bfloat16)
    Eb = Eb_ref[...]
    r = 1.0 / (jnp.sum(Eb.astype(jnp.float32), axis=1, keepdims=True) + 1e-8)
    C_ref[...] = Eb.astype(jnp.float32) * r
    out_ref[...] = jnp.dot(Eb, Hb_ref[...],
                           preferred_element_type=jnp.float32) * r

    AP = jnp.dot(A_ref[...], P_ref[...].astype(jnp.bfloat16),
                 preferred_element_type=jnp.float32)
    Pb = P_ref[pl.ds(i * BR, BR), :]
    contrib = lax.dot_general(Pb, AP, (((0,), (0,)), ((), ())),
                              preferred_element_type=jnp.float32)

    @pl.when(i == 0)
    def _init():
        An_ref[...] = contrib

    @pl.when(i > 0)
    def _acc():
        An_ref[...] += contrib


def _gat_with_an(Z, A, W, a_s, a_r, P):
    N = A.shape[0]
    din, dout = W.shape
    c = P.shape[1]
    out, C, An = pl.pallas_call(
        _gat_an_body,
        grid=(N // BR,),
        in_specs=[
            pl.BlockSpec((N, din), lambda i: (0, 0)),
            pl.BlockSpec((BR, N), lambda i: (i, 0)),
            pl.BlockSpec((din, dout), lambda i: (0, 0)),
            pl.BlockSpec((1, dout), lambda i: (0, 0)),
            pl.BlockSpec((1, dout), lambda i: (0, 0)),
            pl.BlockSpec((N, c), lambda i: (0, 0)),
        ],
        out_specs=(
            pl.BlockSpec((BR, dout), lambda i: (i, 0)),
            pl.BlockSpec((BR, N), lambda i: (i, 0)),
            pl.BlockSpec((c, c), lambda i: (0, 0)),
        ),
        out_shape=(
            jax.ShapeDtypeStruct((N, dout), jnp.float32),
            jax.ShapeDtypeStruct((N, N), jnp.float32),
            jax.ShapeDtypeStruct((c, c), jnp.float32),
        ),
        scratch_shapes=[
            pltpu.VMEM((N, dout), jnp.float32),
            pltpu.VMEM((N, dout), jnp.bfloat16),
            pltpu.VMEM((N, 1), jnp.float32),
            pltpu.VMEM((1, N), jnp.float32),
            pltpu.VMEM((BR, N), jnp.bfloat16),
        ],
    )(Z, A, W, a_s.reshape(1, -1), a_r.reshape(1, -1), P)
    return out, C, An


def _ahat_body(Z_ref, g_ref, b_ref, out_ref):
    i = pl.program_id(0)
    Zb = Z_ref[pl.ds(i * BR, BR), :]
    G = lax.dot_general(Zb, Z_ref[...], (((1,), (1,)), ((), ())),
                        preferred_element_type=jnp.float32)
    mu = jnp.mean(G, axis=1, keepdims=True)
    m2 = jnp.mean(G * G, axis=1, keepdims=True)
    var = m2 - mu * mu
    k = lax.rsqrt(var + 1e-5) * 0.5
    y = (G - mu) * k * g_ref[...] + 0.5 * b_ref[...]
    out_ref[...] = 0.5 * jnp.tanh(y) + 0.5


def _ahat(Z, g, b):
    N = Z.shape[0]
    h = Z.shape[1]
    return pl.pallas_call(
        _ahat_body,
        grid=(N // BR,),
        in_specs=[
            pl.BlockSpec((N, h), lambda i: (0, 0)),
            pl.BlockSpec((1, N), lambda i: (0, 0)),
            pl.BlockSpec((1, N), lambda i: (0, 0)),
        ],
        out_specs=pl.BlockSpec((BR, N), lambda i: (i, 0)),
        out_shape=jax.ShapeDtypeStruct((N, N), jnp.float32),
    )(Z, g.reshape(1, -1), b.reshape(1, -1))


def kernel(X, A, params):
    Ab = _tobf16(A)
    Z = X
    enc_attn = []
    for li in range(3):
        Z, C = _gat(Z, Ab, params['We%d' % li], params['ase%d' % li],
                    params['are%d' % li])
        enc_attn.append(C)

    A_hat = _ahat(Z, params['g_ln'], params['b_ln'])

    P0 = jax.nn.softmax(Z @ params['Wc0'] + params['bc0'], axis=1)
    S0 = jnp.argmax(P0, axis=1)

    dec_attn = []
    Xd, C, An1 = _gat_with_an(Z, Ab, params['Wd0'], params['asd0'],
                              params['ard0'], P0)
    dec_attn.append(C)
    for li in range(1, 3):
        Xd, C = _gat(Xd, Ab, params['Wd%d' % li], params['asd%d' % li],
                     params['ard%d' % li])
        dec_attn.append(C)
    X_hat = Xd

    Xn1 = P0.T @ Z

    P1 = jax.nn.softmax(Xn1 @ params['Wc1'] + params['bc1'], axis=1)
    S1 = jnp.argmax(P1, axis=1)
    Xn2 = P1.T @ Xn1
    An2 = P1.T @ An1 @ P1

    X_all_final = [Z, Xn1, Xn2]
    A_all_final = [A, An1, An2]
    P_all = [P0, P1]
    S_all = [S0, S1]
    return (X_hat, A_hat, X_all_final, A_all_final, P_all, S_all,
            [enc_attn, dec_attn])


# speedup vs baseline: 1.2678x; 1.0765x over previous
import functools

import jax
import jax.numpy as jnp
from jax import lax
from jax.experimental import pallas as pl
from jax.experimental.pallas import tpu as pltpu

BR = 512


def _tobf16_body(A_ref, Ab_ref):
    Ab_ref[...] = A_ref[...].astype(jnp.bfloat16)


def _tobf16(A):
    N = A.shape[0]
    return pl.pallas_call(
        _tobf16_body,
        grid=(N // BR,),
        in_specs=[pl.BlockSpec((BR, N), lambda i: (i, 0))],
        out_specs=pl.BlockSpec((BR, N), lambda i: (i, 0)),
        out_shape=jax.ShapeDtypeStruct((N, N), jnp.bfloat16),
    )(A)


def _gat_body(Z_ref, A_ref, W_ref, as_ref, ar_ref, out_ref, C_ref,
              Hb_ref, f1_ref, f2_ref, Eb_ref):
    i = pl.program_id(0)
    dout = out_ref.shape[1]

    @pl.when(i == 0)
    def _prologue():
        H = jnp.dot(Z_ref[...], W_ref[...], preferred_element_type=jnp.float32)
        n = H.shape[0]
        ones_col = (lax.broadcasted_iota(jnp.int32, (n, 128), 1) == 0)
        Hb_ref[...] = jnp.concatenate(
            [H.astype(jnp.bfloat16), ones_col.astype(jnp.bfloat16)], axis=1)
        f1_ref[...] = 0.5 * lax.dot_general(
            H, as_ref[...], (((1,), (1,)), ((), ())),
            preferred_element_type=jnp.float32)
        f2_ref[...] = 0.5 * lax.dot_general(
            ar_ref[...], H, (((1,), (1,)), ((), ())),
            preferred_element_type=jnp.float32)

    f1b = f1_ref[pl.ds(i * BR, BR), :]
    e = 0.5 * jnp.tanh(f1b + f2_ref[...]) + 0.5
    Eb_ref[...] = (A_ref[...] * e).astype(jnp.bfloat16)
    Eb = Eb_ref[...]
    EHs = jnp.dot(Eb, Hb_ref[...], preferred_element_type=jnp.float32)
    r = 1.0 / (EHs[:, dout:dout + 1] + 1e-8)
    C_ref[...] = Eb.astype(jnp.float32) * r
    out_ref[...] = EHs[:, :dout] * r


def _gat(Z, A, W, a_s, a_r):
    N = A.shape[0]
    din, dout = W.shape
    out, C = pl.pallas_call(
        _gat_body,
        grid=(N // BR,),
        in_specs=[
            pl.BlockSpec((N, din), lambda i: (0, 0)),
            pl.BlockSpec((BR, N), lambda i: (i, 0)),
            pl.BlockSpec((din, dout), lambda i: (0, 0)),
            pl.BlockSpec((1, dout), lambda i: (0, 0)),
            pl.BlockSpec((1, dout), lambda i: (0, 0)),
        ],
        out_specs=(
            pl.BlockSpec((BR, dout), lambda i: (i, 0)),
            pl.BlockSpec((BR, N), lambda i: (i, 0)),
        ),
        out_shape=(
            jax.ShapeDtypeStruct((N, dout), jnp.float32),
            jax.ShapeDtypeStruct((N, N), jnp.float32),
        ),
        scratch_shapes=[
            pltpu.VMEM((N, dout + 128), jnp.bfloat16),
            pltpu.VMEM((N, 1), jnp.float32),
            pltpu.VMEM((1, N), jnp.float32),
            pltpu.VMEM((BR, N), jnp.bfloat16),
        ],
    )(Z, A, W, a_s.reshape(1, -1), a_r.reshape(1, -1))
    return out, C


def _gat_an_body(Z_ref, A_ref, W_ref, as_ref, ar_ref, P_ref,
                 out_ref, C_ref, An_ref, Hb_ref, f1_ref, f2_ref,
                 Eb_ref):
    i = pl.program_id(0)
    dout = out_ref.shape[1]

    @pl.when(i == 0)
    def _prologue():
        H = jnp.dot(Z_ref[...], W_ref[...], preferred_element_type=jnp.float32)
        n = H.shape[0]
        ones_col = (lax.broadcasted_iota(jnp.int32, (n, 128), 1) == 0)
        Hb_ref[...] = jnp.concatenate(
            [H.astype(jnp.bfloat16), ones_col.astype(jnp.bfloat16)], axis=1)
        f1_ref[...] = 0.5 * lax.dot_general(
            H, as_ref[...], (((1,), (1,)), ((), ())),
            preferred_element_type=jnp.float32)
        f2_ref[...] = 0.5 * lax.dot_general(
            ar_ref[...], H, (((1,), (1,)), ((), ())),
            preferred_element_type=jnp.float32)

    f1b = f1_ref[pl.ds(i * BR, BR), :]
    e = 0.5 * jnp.tanh(f1b + f2_ref[...]) + 0.5
    Eb_ref[...] = (A_ref[...] * e).astype(jnp.bfloat16)
    Eb = Eb_ref[...]
    EHs = jnp.dot(Eb, Hb_ref[...], preferred_element_type=jnp.float32)
    r = 1.0 / (EHs[:, dout:dout + 1] + 1e-8)
    C_ref[...] = Eb.astype(jnp.float32) * r
    out_ref[...] = EHs[:, :dout] * r

    AP = jnp.dot(A_ref[...], P_ref[...].astype(jnp.bfloat16),
                 preferred_element_type=jnp.float32)
    Pb = P_ref[pl.ds(i * BR, BR), :]
    contrib = lax.dot_general(Pb, AP, (((0,), (0,)), ((), ())),
                              preferred_element_type=jnp.float32)

    @pl.when(i == 0)
    def _init():
        An_ref[...] = contrib

    @pl.when(i > 0)
    def _acc():
        An_ref[...] += contrib


def _gat_with_an(Z, A, W, a_s, a_r, P):
    N = A.shape[0]
    din, dout = W.shape
    c = P.shape[1]
    out, C, An = pl.pallas_call(
        _gat_an_body,
        grid=(N // BR,),
        in_specs=[
            pl.BlockSpec((N, din), lambda i: (0, 0)),
            pl.BlockSpec((BR, N), lambda i: (i, 0)),
            pl.BlockSpec((din, dout), lambda i: (0, 0)),
            pl.BlockSpec((1, dout), lambda i: (0, 0)),
            pl.BlockSpec((1, dout), lambda i: (0, 0)),
            pl.BlockSpec((N, c), lambda i: (0, 0)),
        ],
        out_specs=(
            pl.BlockSpec((BR, dout), lambda i: (i, 0)),
            pl.BlockSpec((BR, N), lambda i: (i, 0)),
            pl.BlockSpec((c, c), lambda i: (0, 0)),
        ),
        out_shape=(
            jax.ShapeDtypeStruct((N, dout), jnp.float32),
            jax.ShapeDtypeStruct((N, N), jnp.float32),
            jax.ShapeDtypeStruct((c, c), jnp.float32),
        ),
        scratch_shapes=[
            pltpu.VMEM((N, dout + 128), jnp.bfloat16),
            pltpu.VMEM((N, 1), jnp.float32),
            pltpu.VMEM((1, N), jnp.float32),
            pltpu.VMEM((BR, N), jnp.bfloat16),
        ],
    )(Z, A, W, a_s.reshape(1, -1), a_r.reshape(1, -1), P)
    return out, C, An


def _ahat_body(Z_ref, g_ref, b_ref, out_ref):
    i = pl.program_id(0)
    Zb = Z_ref[pl.ds(i * BR, BR), :]
    G = lax.dot_general(Zb, Z_ref[...], (((1,), (1,)), ((), ())),
                        preferred_element_type=jnp.float32)
    mu = jnp.mean(G, axis=1, keepdims=True)
    m2 = jnp.mean(G * G, axis=1, keepdims=True)
    var = m2 - mu * mu
    k = lax.rsqrt(var + 1e-5) * 0.5
    y = (G - mu) * k * g_ref[...] + 0.5 * b_ref[...]
    out_ref[...] = 0.5 * jnp.tanh(y) + 0.5


def _ahat(Z, g, b):
    N = Z.shape[0]
    h = Z.shape[1]
    return pl.pallas_call(
        _ahat_body,
        grid=(N // BR,),
        in_specs=[
            pl.BlockSpec((N, h), lambda i: (0, 0)),
            pl.BlockSpec((1, N), lambda i: (0, 0)),
            pl.BlockSpec((1, N), lambda i: (0, 0)),
        ],
        out_specs=pl.BlockSpec((BR, N), lambda i: (i, 0)),
        out_shape=jax.ShapeDtypeStruct((N, N), jnp.float32),
    )(Z, g.reshape(1, -1), b.reshape(1, -1))


def kernel(X, A, params):
    Ab = _tobf16(A)
    Z = X
    enc_attn = []
    for li in range(3):
        Z, C = _gat(Z, Ab, params['We%d' % li], params['ase%d' % li],
                    params['are%d' % li])
        enc_attn.append(C)

    A_hat = _ahat(Z, params['g_ln'], params['b_ln'])

    P0 = jax.nn.softmax(Z @ params['Wc0'] + params['bc0'], axis=1)
    S0 = jnp.argmax(P0, axis=1)

    dec_attn = []
    Xd, C, An1 = _gat_with_an(Z, Ab, params['Wd0'], params['asd0'],
                              params['ard0'], P0)
    dec_attn.append(C)
    for li in range(1, 3):
        Xd, C = _gat(Xd, Ab, params['Wd%d' % li], params['asd%d' % li],
                     params['ard%d' % li])
        dec_attn.append(C)
    X_hat = Xd

    Xn1 = P0.T @ Z

    P1 = jax.nn.softmax(Xn1 @ params['Wc1'] + params['bc1'], axis=1)
    S1 = jnp.argmax(P1, axis=1)
    Xn2 = P1.T @ Xn1
    An2 = P1.T @ An1 @ P1

    X_all_final = [Z, Xn1, Xn2]
    A_all_final = [A, An1, An2]
    P_all = [P0, P1]
    S_all = [S0, S1]
    return (X_hat, A_hat, X_all_final, A_all_final, P_all, S_all,
            [enc_attn, dec_attn])


# speedup vs baseline: 1.3240x; 1.0443x over previous
import functools

import jax
import jax.numpy as jnp
from jax import lax
from jax.experimental import pallas as pl
from jax.experimental.pallas import tpu as pltpu

BR = 512


def _tobf16_body(A_ref, Ab_ref):
    Ab_ref[...] = A_ref[...].astype(jnp.bfloat16)


def _tobf16(A):
    N = A.shape[0]
    return pl.pallas_call(
        _tobf16_body,
        grid=(N // BR,),
        in_specs=[pl.BlockSpec((BR, N), lambda i: (i, 0))],
        out_specs=pl.BlockSpec((BR, N), lambda i: (i, 0)),
        out_shape=jax.ShapeDtypeStruct((N, N), jnp.bfloat16),
    )(A)


def _gat_body(Z_ref, A_ref, W_ref, as_ref, ar_ref, out_ref, C_ref,
              Hb_ref, f1_ref, f2_ref, Eb_ref):
    i = pl.program_id(0)
    dout = out_ref.shape[1]

    @pl.when(i == 0)
    def _prologue():
        H = jnp.dot(Z_ref[...], W_ref[...], preferred_element_type=jnp.float32)
        n = H.shape[0]
        ones_col = (lax.broadcasted_iota(jnp.int32, (n, 128), 1) == 0)
        Hb_ref[...] = jnp.concatenate(
            [H.astype(jnp.bfloat16), ones_col.astype(jnp.bfloat16)], axis=1)
        f1_ref[...] = 0.5 * lax.dot_general(
            H, as_ref[...], (((1,), (1,)), ((), ())),
            preferred_element_type=jnp.float32)
        f2_ref[...] = 0.5 * lax.dot_general(
            ar_ref[...], H, (((1,), (1,)), ((), ())),
            preferred_element_type=jnp.float32)

    f1b = f1_ref[pl.ds(i * BR, BR), :]
    e = 0.5 * jnp.tanh(f1b + f2_ref[...]) + 0.5
    Eb_ref[...] = (A_ref[...] * e).astype(jnp.bfloat16)
    Eb = Eb_ref[...]
    EHs = jnp.dot(Eb, Hb_ref[...], preferred_element_type=jnp.float32)
    r = 1.0 / (EHs[:, dout:dout + 1] + 1e-8)
    C_ref[...] = Eb.astype(jnp.float32) * r
    out_ref[...] = EHs[:, :dout] * r


def _gat(Z, A, W, a_s, a_r):
    N = A.shape[0]
    din, dout = W.shape
    out, C = pl.pallas_call(
        _gat_body,
        grid=(N // BR,),
        in_specs=[
            pl.BlockSpec((N, din), lambda i: (0, 0)),
            pl.BlockSpec((BR, N), lambda i: (i, 0)),
            pl.BlockSpec((din, dout), lambda i: (0, 0)),
            pl.BlockSpec((1, dout), lambda i: (0, 0)),
            pl.BlockSpec((1, dout), lambda i: (0, 0)),
        ],
        out_specs=(
            pl.BlockSpec((BR, dout), lambda i: (i, 0)),
            pl.BlockSpec((BR, N), lambda i: (i, 0)),
        ),
        out_shape=(
            jax.ShapeDtypeStruct((N, dout), jnp.float32),
            jax.ShapeDtypeStruct((N, N), jnp.float32),
        ),
        scratch_shapes=[
            pltpu.VMEM((N, dout + 128), jnp.bfloat16),
            pltpu.VMEM((N, 1), jnp.float32),
            pltpu.VMEM((1, N), jnp.float32),
            pltpu.VMEM((BR, N), jnp.bfloat16),
        ],
    )(Z, A, W, a_s.reshape(1, -1), a_r.reshape(1, -1))
    return out, C


def _gat_an_body(Z_ref, A_ref, W_ref, as_ref, ar_ref, P_ref,
                 out_ref, C_ref, An_ref, Hb_ref, f1_ref, f2_ref,
                 Eb_ref):
    i = pl.program_id(0)
    dout = out_ref.shape[1]

    @pl.when(i == 0)
    def _prologue():
        H = jnp.dot(Z_ref[...], W_ref[...], preferred_element_type=jnp.float32)
        n = H.shape[0]
        ones_col = (lax.broadcasted_iota(jnp.int32, (n, 128), 1) == 0)
        Hb_ref[...] = jnp.concatenate(
            [H.astype(jnp.bfloat16), ones_col.astype(jnp.bfloat16)], axis=1)
        f1_ref[...] = 0.5 * lax.dot_general(
            H, as_ref[...], (((1,), (1,)), ((), ())),
            preferred_element_type=jnp.float32)
        f2_ref[...] = 0.5 * lax.dot_general(
            ar_ref[...], H, (((1,), (1,)), ((), ())),
            preferred_element_type=jnp.float32)

    f1b = f1_ref[pl.ds(i * BR, BR), :]
    e = 0.5 * jnp.tanh(f1b + f2_ref[...]) + 0.5
    Eb_ref[...] = (A_ref[...] * e).astype(jnp.bfloat16)
    Eb = Eb_ref[...]
    EHs = jnp.dot(Eb, Hb_ref[...], preferred_element_type=jnp.float32)
    r = 1.0 / (EHs[:, dout:dout + 1] + 1e-8)
    C_ref[...] = Eb.astype(jnp.float32) * r
    out_ref[...] = EHs[:, :dout] * r

    AP = jnp.dot(A_ref[...], P_ref[...].astype(jnp.bfloat16),
                 preferred_element_type=jnp.float32)
    Pb = P_ref[pl.ds(i * BR, BR), :]
    contrib = lax.dot_general(Pb, AP, (((0,), (0,)), ((), ())),
                              preferred_element_type=jnp.float32)

    @pl.when(i == 0)
    def _init():
        An_ref[...] = contrib

    @pl.when(i > 0)
    def _acc():
        An_ref[...] += contrib


def _gat_with_an(Z, A, W, a_s, a_r, P):
    N = A.shape[0]
    din, dout = W.shape
    c = P.shape[1]
    out, C, An = pl.pallas_call(
        _gat_an_body,
        grid=(N // BR,),
        in_specs=[
            pl.BlockSpec((N, din), lambda i: (0, 0)),
            pl.BlockSpec((BR, N), lambda i: (i, 0)),
            pl.BlockSpec((din, dout), lambda i: (0, 0)),
            pl.BlockSpec((1, dout), lambda i: (0, 0)),
            pl.BlockSpec((1, dout), lambda i: (0, 0)),
            pl.BlockSpec((N, c), lambda i: (0, 0)),
        ],
        out_specs=(
            pl.BlockSpec((BR, dout), lambda i: (i, 0)),
            pl.BlockSpec((BR, N), lambda i: (i, 0)),
            pl.BlockSpec((c, c), lambda i: (0, 0)),
        ),
        out_shape=(
            jax.ShapeDtypeStruct((N, dout), jnp.float32),
            jax.ShapeDtypeStruct((N, N), jnp.float32),
            jax.ShapeDtypeStruct((c, c), jnp.float32),
        ),
        scratch_shapes=[
            pltpu.VMEM((N, dout + 128), jnp.bfloat16),
            pltpu.VMEM((N, 1), jnp.float32),
            pltpu.VMEM((1, N), jnp.float32),
            pltpu.VMEM((BR, N), jnp.bfloat16),
        ],
    )(Z, A, W, a_s.reshape(1, -1), a_r.reshape(1, -1), P)
    return out, C, An


BRL = 256


def _gat1_body(Z_ref, A_ref, W_ref, as_ref, ar_ref, out_ref, C_ref, Ab_ref,
               Hb_ref, f1_ref, f2_ref, Eb_ref):
    i = pl.program_id(0)
    dout = out_ref.shape[1]

    @pl.when(i == 0)
    def _prologue():
        H = jnp.dot(Z_ref[...], W_ref[...], preferred_element_type=jnp.float32)
        n = H.shape[0]
        ones_col = (lax.broadcasted_iota(jnp.int32, (n, 128), 1) == 0)
        Hb_ref[...] = jnp.concatenate(
            [H.astype(jnp.bfloat16), ones_col.astype(jnp.bfloat16)], axis=1)
        f1_ref[...] = 0.5 * lax.dot_general(
            H, as_ref[...], (((1,), (1,)), ((), ())),
            preferred_element_type=jnp.float32)
        f2_ref[...] = 0.5 * lax.dot_general(
            ar_ref[...], H, (((1,), (1,)), ((), ())),
            preferred_element_type=jnp.float32)

    A_blk = A_ref[...]
    Ab_ref[...] = A_blk.astype(jnp.bfloat16)
    f1b = f1_ref[pl.ds(i * BRL, BRL), :]
    e = 0.5 * jnp.tanh(f1b + f2_ref[...]) + 0.5
    Eb_ref[...] = (A_blk * e).astype(jnp.bfloat16)
    Eb = Eb_ref[...]
    EHs = jnp.dot(Eb, Hb_ref[...], preferred_element_type=jnp.float32)
    r = 1.0 / (EHs[:, dout:dout + 1] + 1e-8)
    C_ref[...] = Eb.astype(jnp.float32) * r
    out_ref[...] = EHs[:, :dout] * r


def _gat1(Z, A, W, a_s, a_r):
    N = A.shape[0]
    din, dout = W.shape
    out, C, Ab = pl.pallas_call(
        _gat1_body,
        grid=(N // BRL,),
        in_specs=[
            pl.BlockSpec((N, din), lambda i: (0, 0)),
            pl.BlockSpec((BRL, N), lambda i: (i, 0)),
            pl.BlockSpec((din, dout), lambda i: (0, 0)),
            pl.BlockSpec((1, dout), lambda i: (0, 0)),
            pl.BlockSpec((1, dout), lambda i: (0, 0)),
        ],
        out_specs=(
            pl.BlockSpec((BRL, dout), lambda i: (i, 0)),
            pl.BlockSpec((BRL, N), lambda i: (i, 0)),
            pl.BlockSpec((BRL, N), lambda i: (i, 0)),
        ),
        out_shape=(
            jax.ShapeDtypeStruct((N, dout), jnp.float32),
            jax.ShapeDtypeStruct((N, N), jnp.float32),
            jax.ShapeDtypeStruct((N, N), jnp.bfloat16),
        ),
        scratch_shapes=[
            pltpu.VMEM((N, dout + 128), jnp.bfloat16),
            pltpu.VMEM((N, 1), jnp.float32),
            pltpu.VMEM((1, N), jnp.float32),
            pltpu.VMEM((BRL, N), jnp.bfloat16),
        ],
    )(Z, A, W, a_s.reshape(1, -1), a_r.reshape(1, -1))
    return out, C, Ab


def _ahat_body(Z_ref, g_ref, b_ref, out_ref):
    i = pl.program_id(0)
    Zb = Z_ref[pl.ds(i * BR, BR), :]
    G = lax.dot_general(Zb, Z_ref[...], (((1,), (1,)), ((), ())),
                        preferred_element_type=jnp.float32)
    mu = jnp.mean(G, axis=1, keepdims=True)
    m2 = jnp.mean(G * G, axis=1, keepdims=True)
    var = m2 - mu * mu
    k = lax.rsqrt(var + 1e-5) * 0.5
    y = (G - mu) * k * g_ref[...] + 0.5 * b_ref[...]
    out_ref[...] = 0.5 * jnp.tanh(y) + 0.5


def _ahat(Z, g, b):
    N = Z.shape[0]
    h = Z.shape[1]
    return pl.pallas_call(
        _ahat_body,
        grid=(N // BR,),
        in_specs=[
            pl.BlockSpec((N, h), lambda i: (0, 0)),
            pl.BlockSpec((1, N), lambda i: (0, 0)),
            pl.BlockSpec((1, N), lambda i: (0, 0)),
        ],
        out_specs=pl.BlockSpec((BR, N), lambda i: (i, 0)),
        out_shape=jax.ShapeDtypeStruct((N, N), jnp.float32),
    )(Z, g.reshape(1, -1), b.reshape(1, -1))


def kernel(X, A, params):
    Z, C1, Ab = _gat1(X, A, params['We0'], params['ase0'], params['are0'])
    enc_attn = [C1]
    for li in range(1, 3):
        Z, C = _gat(Z, Ab, params['We%d' % li], params['ase%d' % li],
                    params['are%d' % li])
        enc_attn.append(C)

    A_hat = _ahat(Z, params['g_ln'], params['b_ln'])

    P0 = jax.nn.softmax(Z @ params['Wc0'] + params['bc0'], axis=1)
    S0 = jnp.argmax(P0, axis=1)

    dec_attn = []
    Xd, C, An1 = _gat_with_an(Z, Ab, params['Wd0'], params['asd0'],
                              params['ard0'], P0)
    dec_attn.append(C)
    for li in range(1, 3):
        Xd, C = _gat(Xd, Ab, params['Wd%d' % li], params['asd%d' % li],
                     params['ard%d' % li])
        dec_attn.append(C)
    X_hat = Xd

    Xn1 = P0.T @ Z

    P1 = jax.nn.softmax(Xn1 @ params['Wc1'] + params['bc1'], axis=1)
    S1 = jnp.argmax(P1, axis=1)
    Xn2 = P1.T @ Xn1
    An2 = P1.T @ An1 @ P1

    X_all_final = [Z, Xn1, Xn2]
    A_all_final = [A, An1, An2]
    P_all = [P0, P1]
    S_all = [S0, S1]
    return (X_hat, A_hat, X_all_final, A_all_final, P_all, S_all,
            [enc_attn, dec_attn])
